# baseline (speedup 1.0000x reference)
import functools

import jax
import jax.numpy as jnp
from jax import lax
from jax.experimental import pallas as pl
from jax.experimental.pallas import tpu as pltpu
from jax.experimental.pallas import tpu_sc as plsc

_CHUNK = 128
_CG = 8
_NCORES = 2
_NSUB = 16
_NW = _NCORES * _NSUB
_LANES = 16
_BLK = 1000


def _stage1_body(mu_ref, x_ref, wT_ref, b_ref, h_ref, x3_ref):
    acc = jnp.dot(x_ref[...], wT_ref[...],
                  preferred_element_type=jnp.float32,
                  precision=lax.Precision.HIGHEST)
    h = jnp.maximum(acc + b_ref[...], 0.0)
    h_ref[...] = h
    mu = mu_ref[...]
    safe = jnp.where(h > 0.0, h, 1.0)
    x3_ref[...] = jnp.where(h > 0.0, jnp.exp(mu * jnp.log(safe)), 0.0)


def _stage2t_body(h_ref, f1T_ref, b1_ref, y1_ref):
    y1_ref[...] = jnp.dot(h_ref[...], f1T_ref[...],
                          preferred_element_type=jnp.float32,
                          precision=lax.Precision.HIGHEST) + b1_ref[...]


def _stage3_body(imu_ref, y1_ref, p_ref, f2T_ref, b2_ref, o_ref):
    p = p_ref[...]
    s = p[0] + p[1]
    imu = imu_ref[...]
    safe = jnp.where(s > 0.0, s, 1.0)
    x2 = jnp.where(s > 0.0, jnp.exp(imu * jnp.log(safe)), 0.0)
    o_ref[...] = (y1_ref[...]
                  + jnp.dot(x2, f2T_ref[...],
                            preferred_element_type=jnp.float32,
                            precision=lax.Precision.HIGHEST)
                  + b2_ref[...])


_DEPTH = 3


def _make_sc_scatter(n_pad, d, e):
    n_chunks = e // _CHUNK
    steps = (n_chunks + _DEPTH * _NW - 1) // (_DEPTH * _NW)
    rows_per_sub = n_pad // _NSUB
    mesh = plsc.VectorSubcoreMesh(core_axis_name="c", subcore_axis_name="s")

    idx_scratch = [pltpu.VMEM((_CHUNK,), jnp.int32) for _ in range(2 * _DEPTH)]
    row_scratch = [pltpu.VMEM((_CHUNK, d), jnp.float32) for _ in range(_DEPTH)]
    sem_scratch = [pltpu.SemaphoreType.DMA for _ in range(2 * _DEPTH)]

    @functools.partial(
        pl.kernel,
        mesh=mesh,
        out_type=jax.ShapeDtypeStruct((_NCORES * n_pad, d), jnp.float32),
        scratch_types=idx_scratch + row_scratch
        + [pltpu.VMEM_SHARED((n_pad, d), jnp.float32)]
        + sem_scratch,
    )
    def scatter_kernel(src_hbm, dst_hbm, x3_hbm, out_hbm, *scratch):
        src_vs = scratch[0:_DEPTH]
        dst_vs = scratch[_DEPTH:2 * _DEPTH]
        rows_vs = scratch[2 * _DEPTH:3 * _DEPTH]
        acc_sh = scratch[3 * _DEPTH]
        gsems = scratch[3 * _DEPTH + 1:4 * _DEPTH + 1]
        ssems = scratch[4 * _DEPTH + 1:]
        c = lax.axis_index("c")
        s = lax.axis_index("s")
        w = s * _NCORES + c

        zrow = jnp.zeros((_LANES,), jnp.float32)

        @pl.loop(0, _CHUNK)
        def _(i):
            @pl.loop(0, d, step=_LANES)
            def _(j):
                rows_vs[0][i, pl.ds(j, _LANES)] = zrow

        @pl.loop(0, rows_per_sub, step=_CG)
        def _(r):
            pltpu.sync_copy(rows_vs[0].at[pl.ds(0, _CG)],
                            acc_sh.at[pl.ds(s * rows_per_sub + r, _CG)])

        plsc.subcore_barrier()

        n_units = n_chunks // _DEPTH

        @pl.loop(0, steps)
        def _(k):
            u = k * _NW + w

            @pl.when(u < n_units)
            def _():
                @pl.when(k > 0)
                def _():
                    for q in range(_DEPTH):
                        pltpu.make_async_copy(rows_vs[q],
                                              acc_sh.at[dst_vs[q]],
                                              ssems[q]).wait()

                j0 = _DEPTH * u
                ih = []
                for q in range(_DEPTH):
                    base = pl.multiple_of((j0 + q) * _CHUNK, _CHUNK)
                    ih.append(pltpu.async_copy(
                        src_hbm.at[pl.ds(base, _CHUNK)], src_vs[q], gsems[q]))
                    ih.append(pltpu.async_copy(
                        dst_hbm.at[pl.ds(base, _CHUNK)], dst_vs[q], gsems[q]))
                gh = []
                for q in range(_DEPTH):
                    ih[2 * q].wait()
                    ih[2 * q + 1].wait()
                    gh.append(pltpu.async_copy(
                        x3_hbm.at[src_vs[q]], rows_vs[q], gsems[q]))
                for q in range(_DEPTH):
                    gh[q].wait()
                    pltpu.async_copy(rows_vs[q], acc_sh.at[dst_vs[q]],
                                     ssems[q], add=True)

        for q in range(_DEPTH):
            pltpu.make_async_copy(rows_vs[q], acc_sh.at[dst_vs[q]],
                                  ssems[q]).wait()

        plsc.subcore_barrier()

        @pl.loop(0, rows_per_sub, step=_CG)
        def _(r):
            row = s * rows_per_sub + r
            pltpu.sync_copy(acc_sh.at[pl.ds(row, _CG)],
                            out_hbm.at[pl.ds(c * n_pad + row, _CG)])

    return scatter_kernel


def kernel(x, edge_index, pool_W, pool_b, fc1_W, fc1_b, fc2_W, fc2_b, mu):
    n, d_in = x.shape
    d_pool = pool_W.shape[0]
    d_out = fc1_W.shape[0]
    e = edge_index.shape[1]
    pad_unit = _NSUB * _CG
    n_pad = ((n + pad_unit - 1) // pad_unit) * pad_unit
    grid = n // _BLK

    mu_f = jnp.asarray(mu, jnp.float32).reshape(1, 1)
    mu_row = jnp.broadcast_to(mu_f, (1, d_pool))
    imu_row = jnp.broadcast_to(1.0 / mu_f, (1, d_pool))

    h, x3 = pl.pallas_call(
        _stage1_body,
        grid=(grid,),
        in_specs=[
            pl.BlockSpec((1, d_pool), lambda i: (0, 0)),
            pl.BlockSpec((_BLK, d_in), lambda i: (i, 0)),
            pl.BlockSpec((d_in, d_pool), lambda i: (0, 0)),
            pl.BlockSpec((1, d_pool), lambda i: (0, 0)),
        ],
        out_specs=[
            pl.BlockSpec((_BLK, d_pool), lambda i: (i, 0)),
            pl.BlockSpec((_BLK, d_pool), lambda i: (i, 0)),
        ],
        out_shape=[
            jax.ShapeDtypeStruct((n, d_pool), jnp.float32),
            jax.ShapeDtypeStruct((n, d_pool), jnp.float32),
        ],
    )(mu_row, x, pool_W.T, pool_b.reshape(1, -1))

    dst = edge_index[0]
    src = edge_index[1]
    unit = _DEPTH * _CHUNK
    e_pad = ((e + unit - 1) // unit) * unit
    pad = e_pad - e
    if pad:
        src = jnp.concatenate([src, jnp.zeros((pad,), jnp.int32)])
        junk = n + (jnp.arange(pad, dtype=jnp.int32) % (n_pad - n))
        dst = jnp.concatenate([dst, junk])
    agg_flat = _make_sc_scatter(n_pad, d_pool, e_pad)(src, dst, x3)
    agg3 = agg_flat.reshape(_NCORES, n_pad, d_pool)

    y1 = pl.pallas_call(
        _stage2t_body,
        grid=(grid,),
        in_specs=[
            pl.BlockSpec((_BLK, d_pool), lambda i: (i, 0)),
            pl.BlockSpec((d_pool, d_out), lambda i: (0, 0)),
            pl.BlockSpec((1, d_out), lambda i: (0, 0)),
        ],
        out_specs=pl.BlockSpec((_BLK, d_out), lambda i: (i, 0)),
        out_shape=jax.ShapeDtypeStruct((n, d_out), jnp.float32),
    )(h, fc1_W.T, fc1_b.reshape(1, -1))

    out = pl.pallas_call(
        _stage3_body,
        grid=(grid,),
        in_specs=[
            pl.BlockSpec((1, d_pool), lambda i: (0, 0)),
            pl.BlockSpec((_BLK, d_out), lambda i: (i, 0)),
            pl.BlockSpec((_NCORES, _BLK, d_pool), lambda i: (0, i, 0)),
            pl.BlockSpec((d_pool, d_out), lambda i: (0, 0)),
            pl.BlockSpec((1, d_out), lambda i: (0, 0)),
        ],
        out_specs=pl.BlockSpec((_BLK, d_out), lambda i: (i, 0)),
        out_shape=jax.ShapeDtypeStruct((n, d_out), jnp.float32),
    )(imu_row, y1, agg3, fc2_W.T, fc2_b.reshape(1, -1))

    return out


# speedup vs baseline: 1.2733x; 1.2733x over previous
import functools

import jax
import jax.numpy as jnp
from jax import lax
from jax.experimental import pallas as pl
from jax.experimental.pallas import tpu as pltpu
from jax.experimental.pallas import tpu_sc as plsc

_CHUNK = 120
_CG = 64
_NCORES = 2
_NSUB = 16
_NW = _NCORES * _NSUB
_LANES = 16
_BLK = 1000


def _stage1_body(mu_ref, x_ref, wT_ref, b_ref, h_ref, x3_ref):
    acc = jnp.dot(x_ref[...], wT_ref[...],
                  preferred_element_type=jnp.float32,
                  precision=lax.Precision.HIGHEST)
    h = jnp.maximum(acc + b_ref[...], 0.0)
    h_ref[...] = h
    mu = mu_ref[...]
    safe = jnp.where(h > 0.0, h, 1.0)
    x3_ref[...] = jnp.where(h > 0.0, jnp.exp(mu * jnp.log(safe)), 0.0)


def _stage2t_body(h_ref, f1T_ref, b1_ref, y1_ref):
    y1_ref[...] = jnp.dot(h_ref[...], f1T_ref[...],
                          preferred_element_type=jnp.float32,
                          precision=lax.Precision.HIGHEST) + b1_ref[...]


def _stage3_body(imu_ref, y1_ref, p_ref, f2T_ref, b2_ref, o_ref):
    p = p_ref[...]
    s = p[0] + p[1]
    imu = imu_ref[...]
    safe = jnp.where(s > 0.0, s, 1.0)
    x2 = jnp.where(s > 0.0, jnp.exp(imu * jnp.log(safe)), 0.0)
    o_ref[...] = (y1_ref[...]
                  + jnp.dot(x2, f2T_ref[...],
                            preferred_element_type=jnp.float32,
                            precision=lax.Precision.HIGHEST)
                  + b2_ref[...])


_DEPTH = 3


def _make_sc_scatter(n_pad, d, e):
    n_chunks = e // _CHUNK
    steps = (n_chunks + _DEPTH * _NW - 1) // (_DEPTH * _NW)
    rows_per_sub = n_pad // _NSUB
    mesh = plsc.VectorSubcoreMesh(core_axis_name="c", subcore_axis_name="s")

    idx_scratch = [pltpu.VMEM((_CHUNK,), jnp.int32) for _ in range(2 * _DEPTH)]
    row_scratch = [pltpu.VMEM((_CHUNK, d), jnp.float32) for _ in range(_DEPTH)]
    sem_scratch = [pltpu.SemaphoreType.DMA for _ in range(2 * _DEPTH)]

    @functools.partial(
        pl.kernel,
        mesh=mesh,
        out_type=jax.ShapeDtypeStruct((_NCORES * n_pad, d), jnp.float32),
        scratch_types=idx_scratch + row_scratch
        + [pltpu.VMEM_SHARED((n_pad, d), jnp.float32)]
        + sem_scratch,
    )
    def scatter_kernel(src_hbm, dst_hbm, x3_hbm, out_hbm, *scratch):
        src_vs = scratch[0:_DEPTH]
        dst_vs = scratch[_DEPTH:2 * _DEPTH]
        rows_vs = scratch[2 * _DEPTH:3 * _DEPTH]
        acc_sh = scratch[3 * _DEPTH]
        gsems = scratch[3 * _DEPTH + 1:4 * _DEPTH + 1]
        ssems = scratch[4 * _DEPTH + 1:]
        c = lax.axis_index("c")
        s = lax.axis_index("s")
        w = s * _NCORES + c

        zrow = jnp.zeros((_LANES,), jnp.float32)

        @pl.loop(0, _CHUNK)
        def _(i):
            @pl.loop(0, d, step=_LANES)
            def _(j):
                rows_vs[0][i, pl.ds(j, _LANES)] = zrow

        @pl.loop(0, rows_per_sub, step=_CG)
        def _(r):
            pltpu.sync_copy(rows_vs[0].at[pl.ds(0, _CG)],
                            acc_sh.at[pl.ds(s * rows_per_sub + r, _CG)])

        plsc.subcore_barrier()

        n_units = n_chunks // _DEPTH

        @pl.loop(0, steps)
        def _(k):
            u = k * _NW + w

            @pl.when(u < n_units)
            def _():
                @pl.when(k > 0)
                def _():
                    for q in range(_DEPTH):
                        pltpu.make_async_copy(rows_vs[q],
                                              acc_sh.at[dst_vs[q]],
                                              ssems[q]).wait()

                j0 = _DEPTH * u
                ih = []
                for q in range(_DEPTH):
                    base = pl.multiple_of((j0 + q) * _CHUNK, _CHUNK)
                    ih.append(pltpu.async_copy(
                        src_hbm.at[pl.ds(base, _CHUNK)], src_vs[q], gsems[q]))
                    ih.append(pltpu.async_copy(
                        dst_hbm.at[pl.ds(base, _CHUNK)], dst_vs[q], gsems[q]))
                gh = []
                for q in range(_DEPTH):
                    ih[2 * q].wait()
                    ih[2 * q + 1].wait()
                    gh.append(pltpu.async_copy(
                        x3_hbm.at[src_vs[q]], rows_vs[q], gsems[q]))
                for q in range(_DEPTH):
                    gh[q].wait()
                    pltpu.async_copy(rows_vs[q], acc_sh.at[dst_vs[q]],
                                     ssems[q], add=True)

        for q in range(_DEPTH):
            pltpu.make_async_copy(rows_vs[q], acc_sh.at[dst_vs[q]],
                                  ssems[q]).wait()

        plsc.subcore_barrier()

        row0 = s * rows_per_sub
        pltpu.sync_copy(acc_sh.at[pl.ds(row0, rows_per_sub)],
                        out_hbm.at[pl.ds(c * n_pad + row0, rows_per_sub)])

    return scatter_kernel


def kernel(x, edge_index, pool_W, pool_b, fc1_W, fc1_b, fc2_W, fc2_b, mu):
    n, d_in = x.shape
    d_pool = pool_W.shape[0]
    d_out = fc1_W.shape[0]
    e = edge_index.shape[1]
    pad_unit = _NSUB * _CG
    n_pad = ((n + pad_unit - 1) // pad_unit) * pad_unit
    grid = n // _BLK

    mu_f = jnp.asarray(mu, jnp.float32).reshape(1, 1)
    mu_row = jnp.broadcast_to(mu_f, (1, d_pool))
    imu_row = jnp.broadcast_to(1.0 / mu_f, (1, d_pool))

    h, x3 = pl.pallas_call(
        _stage1_body,
        grid=(grid,),
        in_specs=[
            pl.BlockSpec((1, d_pool), lambda i: (0, 0)),
            pl.BlockSpec((_BLK, d_in), lambda i: (i, 0)),
            pl.BlockSpec((d_in, d_pool), lambda i: (0, 0)),
            pl.BlockSpec((1, d_pool), lambda i: (0, 0)),
        ],
        out_specs=[
            pl.BlockSpec((_BLK, d_pool), lambda i: (i, 0)),
            pl.BlockSpec((_BLK, d_pool), lambda i: (i, 0)),
        ],
        out_shape=[
            jax.ShapeDtypeStruct((n, d_pool), jnp.float32),
            jax.ShapeDtypeStruct((n, d_pool), jnp.float32),
        ],
    )(mu_row, x, pool_W.T, pool_b.reshape(1, -1))

    dst = edge_index[0]
    src = edge_index[1]
    unit = _DEPTH * _CHUNK
    e_pad = ((e + unit - 1) // unit) * unit
    pad = e_pad - e
    if pad:
        src = jnp.concatenate([src, jnp.zeros((pad,), jnp.int32)])
        junk = n + (jnp.arange(pad, dtype=jnp.int32) % (n_pad - n))
        dst = jnp.concatenate([dst, junk])
    agg_flat = _make_sc_scatter(n_pad, d_pool, e_pad)(src, dst, x3)
    agg3 = agg_flat.reshape(_NCORES, n_pad, d_pool)

    y1 = pl.pallas_call(
        _stage2t_body,
        grid=(grid,),
        in_specs=[
            pl.BlockSpec((_BLK, d_pool), lambda i: (i, 0)),
            pl.BlockSpec((d_pool, d_out), lambda i: (0, 0)),
            pl.BlockSpec((1, d_out), lambda i: (0, 0)),
        ],
        out_specs=pl.BlockSpec((_BLK, d_out), lambda i: (i, 0)),
        out_shape=jax.ShapeDtypeStruct((n, d_out), jnp.float32),
    )(h, fc1_W.T, fc1_b.reshape(1, -1))

    out = pl.pallas_call(
        _stage3_body,
        grid=(grid,),
        in_specs=[
            pl.BlockSpec((1, d_pool), lambda i: (0, 0)),
            pl.BlockSpec((_BLK, d_out), lambda i: (i, 0)),
            pl.BlockSpec((_NCORES, _BLK, d_pool), lambda i: (0, i, 0)),
            pl.BlockSpec((d_pool, d_out), lambda i: (0, 0)),
            pl.BlockSpec((1, d_out), lambda i: (0, 0)),
        ],
        out_specs=pl.BlockSpec((_BLK, d_out), lambda i: (i, 0)),
        out_shape=jax.ShapeDtypeStruct((n, d_out), jnp.float32),
    )(imu_row, y1, agg3, fc2_W.T, fc2_b.reshape(1, -1))

    return out


# speedup vs baseline: 1.2777x; 1.0035x over previous
import functools

import jax
import jax.numpy as jnp
from jax import lax
from jax.experimental import pallas as pl
from jax.experimental.pallas import tpu as pltpu
from jax.experimental.pallas import tpu_sc as plsc

_CHUNK = 120
_CG = 64
_NCORES = 2
_NSUB = 16
_NW = _NCORES * _NSUB
_LANES = 16
_BLK = 1000


def _stage1_body(mu_ref, x_ref, wT_ref, b_ref, h_ref, x3_ref):
    acc = jnp.dot(x_ref[...], wT_ref[...],
                  preferred_element_type=jnp.float32,
                  precision=lax.Precision.HIGHEST)
    h = jnp.maximum(acc + b_ref[...], 0.0)
    h_ref[...] = h
    mu = mu_ref[...]
    safe = jnp.where(h > 0.0, h, 1.0)
    x3_ref[...] = jnp.where(h > 0.0, jnp.exp(mu * jnp.log(safe)), 0.0)


def _stage2t_body(h_ref, f1T_ref, b1_ref, y1_ref):
    y1_ref[...] = jnp.dot(h_ref[...], f1T_ref[...],
                          preferred_element_type=jnp.float32,
                          precision=lax.Precision.HIGHEST) + b1_ref[...]


def _stage3_body(imu_ref, y1_ref, p_ref, f2T_ref, b2_ref, o_ref):
    p = p_ref[...]
    s = p[0] + p[1]
    imu = imu_ref[...]
    safe = jnp.where(s > 0.0, s, 1.0)
    x2 = jnp.where(s > 0.0, jnp.exp(imu * jnp.log(safe)), 0.0)
    o_ref[...] = (y1_ref[...]
                  + jnp.dot(x2, f2T_ref[...],
                            preferred_element_type=jnp.float32,
                            precision=lax.Precision.HIGHEST)
                  + b2_ref[...])


_DEPTH = 3


def _make_sc_scatter(n_pad, d, e):
    n_chunks = e // _CHUNK
    steps = (n_chunks + _DEPTH * _NW - 1) // (_DEPTH * _NW)
    rows_per_sub = n_pad // _NSUB
    mesh = plsc.VectorSubcoreMesh(core_axis_name="c", subcore_axis_name="s")

    idx_scratch = [pltpu.VMEM((_CHUNK,), jnp.int32) for _ in range(2 * _DEPTH)]
    row_scratch = [pltpu.VMEM((_CHUNK, d), jnp.float32) for _ in range(_DEPTH)]
    sem_scratch = [pltpu.SemaphoreType.DMA for _ in range(2 * _DEPTH)]

    @functools.partial(
        pl.kernel,
        mesh=mesh,
        out_type=jax.ShapeDtypeStruct((_NCORES * n_pad, d), jnp.float32),
        scratch_types=idx_scratch + row_scratch
        + [pltpu.VMEM_SHARED((n_pad, d), jnp.float32)]
        + sem_scratch,
    )
    def scatter_kernel(src_hbm, dst_hbm, x3_hbm, out_hbm, *scratch):
        src_vs = scratch[0:_DEPTH]
        dst_vs = scratch[_DEPTH:2 * _DEPTH]
        rows_vs = scratch[2 * _DEPTH:3 * _DEPTH]
        acc_sh = scratch[3 * _DEPTH]
        gsems = scratch[3 * _DEPTH + 1:4 * _DEPTH + 1]
        ssems = scratch[4 * _DEPTH + 1:]
        c = lax.axis_index("c")
        s = lax.axis_index("s")
        w = s * _NCORES + c

        zrow = jnp.zeros((_LANES,), jnp.float32)

        @pl.loop(0, _CHUNK)
        def _(i):
            @pl.loop(0, d, step=_LANES)
            def _(j):
                rows_vs[0][i, pl.ds(j, _LANES)] = zrow

        zh = []
        for t in range(rows_per_sub // _CG):
            zh.append(pltpu.async_copy(
                rows_vs[0].at[pl.ds(0, _CG)],
                acc_sh.at[pl.ds(s * rows_per_sub + t * _CG, _CG)],
                gsems[t % _DEPTH]))
        for h in zh:
            h.wait()

        plsc.subcore_barrier()

        n_units = n_chunks // _DEPTH

        @pl.loop(0, steps)
        def _(k):
            u = k * _NW + w

            @pl.when(u < n_units)
            def _():
                @pl.when(k > 0)
                def _():
                    for q in range(_DEPTH):
                        pltpu.make_async_copy(rows_vs[q],
                                              acc_sh.at[dst_vs[q]],
                                              ssems[q]).wait()

                j0 = _DEPTH * u
                ih = []
                for q in range(_DEPTH):
                    base = pl.multiple_of((j0 + q) * _CHUNK, _CHUNK)
                    ih.append(pltpu.async_copy(
                        src_hbm.at[pl.ds(base, _CHUNK)], src_vs[q], gsems[q]))
                    ih.append(pltpu.async_copy(
                        dst_hbm.at[pl.ds(base, _CHUNK)], dst_vs[q], gsems[q]))
                gh = []
                for q in range(_DEPTH):
                    ih[2 * q].wait()
                    ih[2 * q + 1].wait()
                    gh.append(pltpu.async_copy(
                        x3_hbm.at[src_vs[q]], rows_vs[q], gsems[q]))
                for q in range(_DEPTH):
                    gh[q].wait()
                    pltpu.async_copy(rows_vs[q], acc_sh.at[dst_vs[q]],
                                     ssems[q], add=True)

        for q in range(_DEPTH):
            pltpu.make_async_copy(rows_vs[q], acc_sh.at[dst_vs[q]],
                                  ssems[q]).wait()

        plsc.subcore_barrier()

        row0 = s * rows_per_sub
        pltpu.sync_copy(acc_sh.at[pl.ds(row0, rows_per_sub)],
                        out_hbm.at[pl.ds(c * n_pad + row0, rows_per_sub)])

    return scatter_kernel


def kernel(x, edge_index, pool_W, pool_b, fc1_W, fc1_b, fc2_W, fc2_b, mu):
    n, d_in = x.shape
    d_pool = pool_W.shape[0]
    d_out = fc1_W.shape[0]
    e = edge_index.shape[1]
    pad_unit = _NSUB * _CG
    n_pad = ((n + pad_unit - 1) // pad_unit) * pad_unit
    grid = n // _BLK

    mu_f = jnp.asarray(mu, jnp.float32).reshape(1, 1)
    mu_row = jnp.broadcast_to(mu_f, (1, d_pool))
    imu_row = jnp.broadcast_to(1.0 / mu_f, (1, d_pool))

    h, x3 = pl.pallas_call(
        _stage1_body,
        grid=(grid,),
        in_specs=[
            pl.BlockSpec((1, d_pool), lambda i: (0, 0)),
            pl.BlockSpec((_BLK, d_in), lambda i: (i, 0)),
            pl.BlockSpec((d_in, d_pool), lambda i: (0, 0)),
            pl.BlockSpec((1, d_pool), lambda i: (0, 0)),
        ],
        out_specs=[
            pl.BlockSpec((_BLK, d_pool), lambda i: (i, 0)),
            pl.BlockSpec((_BLK, d_pool), lambda i: (i, 0)),
        ],
        out_shape=[
            jax.ShapeDtypeStruct((n, d_pool), jnp.float32),
            jax.ShapeDtypeStruct((n, d_pool), jnp.float32),
        ],
    )(mu_row, x, pool_W.T, pool_b.reshape(1, -1))

    dst = edge_index[0]
    src = edge_index[1]
    unit = _DEPTH * _CHUNK
    e_pad = ((e + unit - 1) // unit) * unit
    pad = e_pad - e
    if pad:
        src = jnp.concatenate([src, jnp.zeros((pad,), jnp.int32)])
        junk = n + (jnp.arange(pad, dtype=jnp.int32) % (n_pad - n))
        dst = jnp.concatenate([dst, junk])
    agg_flat = _make_sc_scatter(n_pad, d_pool, e_pad)(src, dst, x3)
    agg3 = agg_flat.reshape(_NCORES, n_pad, d_pool)

    y1 = pl.pallas_call(
        _stage2t_body,
        grid=(grid,),
        in_specs=[
            pl.BlockSpec((_BLK, d_pool), lambda i: (i, 0)),
            pl.BlockSpec((d_pool, d_out), lambda i: (0, 0)),
            pl.BlockSpec((1, d_out), lambda i: (0, 0)),
        ],
        out_specs=pl.BlockSpec((_BLK, d_out), lambda i: (i, 0)),
        out_shape=jax.ShapeDtypeStruct((n, d_out), jnp.float32),
    )(h, fc1_W.T, fc1_b.reshape(1, -1))

    out = pl.pallas_call(
        _stage3_body,
        grid=(grid,),
        in_specs=[
            pl.BlockSpec((1, d_pool), lambda i: (0, 0)),
            pl.BlockSpec((_BLK, d_out), lambda i: (i, 0)),
            pl.BlockSpec((_NCORES, _BLK, d_pool), lambda i: (0, i, 0)),
            pl.BlockSpec((d_pool, d_out), lambda i: (0, 0)),
            pl.BlockSpec((1, d_out), lambda i: (0, 0)),
        ],
        out_specs=pl.BlockSpec((_BLK, d_out), lambda i: (i, 0)),
        out_shape=jax.ShapeDtypeStruct((n, d_out), jnp.float32),
    )(imu_row, y1, agg3, fc2_W.T, fc2_b.reshape(1, -1))

    return out


# speedup vs baseline: 1.3442x; 1.0520x over previous
import functools

import jax
import jax.numpy as jnp
from jax import lax
from jax.experimental import pallas as pl
from jax.experimental.pallas import tpu as pltpu
from jax.experimental.pallas import tpu_sc as plsc

_CHUNK = 120
_CG = 64
_NCORES = 2
_NSUB = 16
_NW = _NCORES * _NSUB
_LANES = 16
_BLK = 1000


def _stage1_body(mu_ref, x_ref, wT_ref, b_ref, h_ref, x3_ref):
    acc = jnp.dot(x_ref[...], wT_ref[...],
                  preferred_element_type=jnp.float32,
                  precision=lax.Precision.HIGHEST)
    h = jnp.maximum(acc + b_ref[...], 0.0)
    h_ref[...] = h
    mu = mu_ref[...]
    safe = jnp.where(h > 0.0, h, 1.0)
    x3_ref[...] = jnp.where(h > 0.0, jnp.exp(mu * jnp.log(safe)), 0.0)


def _stage2t_body(h_ref, f1T_ref, b1_ref, y1_ref):
    y1_ref[...] = jnp.dot(h_ref[...], f1T_ref[...],
                          preferred_element_type=jnp.float32,
                          precision=lax.Precision.HIGHEST) + b1_ref[...]


def _stage3_body(imu_ref, y1_ref, p_ref, f2T_ref, b2_ref, o_ref):
    p = p_ref[...]
    s = p[0] + p[1]
    imu = imu_ref[...]
    safe = jnp.where(s > 0.0, s, 1.0)
    x2 = jnp.where(s > 0.0, jnp.exp(imu * jnp.log(safe)), 0.0)
    o_ref[...] = (y1_ref[...]
                  + jnp.dot(x2, f2T_ref[...],
                            preferred_element_type=jnp.float32,
                            precision=lax.Precision.HIGHEST)
                  + b2_ref[...])


_DEPTH = 3


def _make_sc_scatter(n_pad, d, e):
    n_chunks = e // _CHUNK
    steps = (n_chunks // _DEPTH + _NW - 1) // _NW
    steps += steps % 2
    rows_per_sub = n_pad // _NSUB
    mesh = plsc.VectorSubcoreMesh(core_axis_name="c", subcore_axis_name="s")

    idx_scratch = [pltpu.VMEM((_CHUNK,), jnp.int32) for _ in range(4 * _DEPTH)]
    row_scratch = [pltpu.VMEM((_CHUNK, d), jnp.float32) for _ in range(_DEPTH)]
    sem_scratch = [pltpu.SemaphoreType.DMA for _ in range(2 * _DEPTH + 2)]

    @functools.partial(
        pl.kernel,
        mesh=mesh,
        out_type=jax.ShapeDtypeStruct((_NCORES * n_pad, d), jnp.float32),
        scratch_types=idx_scratch + row_scratch
        + [pltpu.VMEM_SHARED((n_pad, d), jnp.float32)]
        + sem_scratch,
    )
    def scatter_kernel(src_hbm, dst_hbm, x3_hbm, out_hbm, *scratch):
        src_vs = (scratch[0:_DEPTH], scratch[_DEPTH:2 * _DEPTH])
        dst_vs = (scratch[2 * _DEPTH:3 * _DEPTH],
                  scratch[3 * _DEPTH:4 * _DEPTH])
        rows_vs = scratch[4 * _DEPTH:5 * _DEPTH]
        acc_sh = scratch[5 * _DEPTH]
        gsems = scratch[5 * _DEPTH + 1:6 * _DEPTH + 1]
        ssems = scratch[6 * _DEPTH + 1:7 * _DEPTH + 1]
        isems = scratch[7 * _DEPTH + 1:]
        c = lax.axis_index("c")
        s = lax.axis_index("s")
        w = s * _NCORES + c

        zrow = jnp.zeros((_LANES,), jnp.float32)

        @pl.loop(0, _CHUNK)
        def _(i):
            @pl.loop(0, d, step=_LANES)
            def _(j):
                rows_vs[0][i, pl.ds(j, _LANES)] = zrow

        zh = []
        for t in range(rows_per_sub // _CG):
            zh.append(pltpu.async_copy(
                rows_vs[0].at[pl.ds(0, _CG)],
                acc_sh.at[pl.ds(s * rows_per_sub + t * _CG, _CG)],
                gsems[t % _DEPTH]))
        for h in zh:
            h.wait()

        plsc.subcore_barrier()

        n_units = n_chunks // _DEPTH

        def fire_idx(u, pset):
            j0 = _DEPTH * u
            for q in range(_DEPTH):
                base = pl.multiple_of((j0 + q) * _CHUNK, _CHUNK)
                pltpu.async_copy(src_hbm.at[pl.ds(base, _CHUNK)],
                                 src_vs[pset][q], isems[pset])
                pltpu.async_copy(dst_hbm.at[pl.ds(base, _CHUNK)],
                                 dst_vs[pset][q], isems[pset])

        def wait_idx(u, pset):
            j0 = _DEPTH * u
            for q in range(_DEPTH):
                base = pl.multiple_of((j0 + q) * _CHUNK, _CHUNK)
                pltpu.make_async_copy(src_hbm.at[pl.ds(base, _CHUNK)],
                                      src_vs[pset][q], isems[pset]).wait()
                pltpu.make_async_copy(dst_hbm.at[pl.ds(base, _CHUNK)],
                                      dst_vs[pset][q], isems[pset]).wait()

        fire_idx(w, 0)

        @pl.loop(0, steps, step=2)
        def _(k0):
            for pset in (0, 1):
                k = k0 + pset
                u = k * _NW + w

                @pl.when(u < n_units)
                def _(k=k, u=u, pset=pset):
                    wait_idx(u, pset)

                    if pset == 0:
                        @pl.when(k > 0)
                        def _():
                            for q in range(_DEPTH):
                                pltpu.make_async_copy(
                                    rows_vs[q], acc_sh.at[dst_vs[1][q]],
                                    ssems[q]).wait()
                    else:
                        for q in range(_DEPTH):
                            pltpu.make_async_copy(
                                rows_vs[q], acc_sh.at[dst_vs[0][q]],
                                ssems[q]).wait()

                    gh = []
                    for q in range(_DEPTH):
                        gh.append(pltpu.async_copy(
                            x3_hbm.at[src_vs[pset][q]], rows_vs[q],
                            gsems[q]))

                    @pl.when(u + _NW < n_units)
                    def _():
                        fire_idx(u + _NW, 1 - pset)

                    for q in range(_DEPTH):
                        gh[q].wait()
                        pltpu.async_copy(rows_vs[q],
                                         acc_sh.at[dst_vs[pset][q]],
                                         ssems[q], add=True)

        for q in range(_DEPTH):
            pltpu.make_async_copy(rows_vs[q], acc_sh.at[dst_vs[0][q]],
                                  ssems[q]).wait()

        plsc.subcore_barrier()

        row0 = s * rows_per_sub
        pltpu.sync_copy(acc_sh.at[pl.ds(row0, rows_per_sub)],
                        out_hbm.at[pl.ds(c * n_pad + row0, rows_per_sub)])

    return scatter_kernel


def kernel(x, edge_index, pool_W, pool_b, fc1_W, fc1_b, fc2_W, fc2_b, mu):
    n, d_in = x.shape
    d_pool = pool_W.shape[0]
    d_out = fc1_W.shape[0]
    e = edge_index.shape[1]
    pad_unit = _NSUB * _CG
    n_pad = ((n + pad_unit - 1) // pad_unit) * pad_unit
    grid = n // _BLK

    mu_f = jnp.asarray(mu, jnp.float32).reshape(1, 1)
    mu_row = jnp.broadcast_to(mu_f, (1, d_pool))
    imu_row = jnp.broadcast_to(1.0 / mu_f, (1, d_pool))

    h, x3 = pl.pallas_call(
        _stage1_body,
        grid=(grid,),
        in_specs=[
            pl.BlockSpec((1, d_pool), lambda i: (0, 0)),
            pl.BlockSpec((_BLK, d_in), lambda i: (i, 0)),
            pl.BlockSpec((d_in, d_pool), lambda i: (0, 0)),
            pl.BlockSpec((1, d_pool), lambda i: (0, 0)),
        ],
        out_specs=[
            pl.BlockSpec((_BLK, d_pool), lambda i: (i, 0)),
            pl.BlockSpec((_BLK, d_pool), lambda i: (i, 0)),
        ],
        out_shape=[
            jax.ShapeDtypeStruct((n, d_pool), jnp.float32),
            jax.ShapeDtypeStruct((n, d_pool), jnp.float32),
        ],
    )(mu_row, x, pool_W.T, pool_b.reshape(1, -1))

    dst = edge_index[0]
    src = edge_index[1]
    unit = _DEPTH * _CHUNK
    e_pad = ((e + unit - 1) // unit) * unit
    pad = e_pad - e
    if pad:
        src = jnp.concatenate([src, jnp.zeros((pad,), jnp.int32)])
        junk = n + (jnp.arange(pad, dtype=jnp.int32) % (n_pad - n))
        dst = jnp.concatenate([dst, junk])
    agg_flat = _make_sc_scatter(n_pad, d_pool, e_pad)(src, dst, x3)
    agg3 = agg_flat.reshape(_NCORES, n_pad, d_pool)

    y1 = pl.pallas_call(
        _stage2t_body,
        grid=(grid,),
        in_specs=[
            pl.BlockSpec((_BLK, d_pool), lambda i: (i, 0)),
            pl.BlockSpec((d_pool, d_out), lambda i: (0, 0)),
            pl.BlockSpec((1, d_out), lambda i: (0, 0)),
        ],
        out_specs=pl.BlockSpec((_BLK, d_out), lambda i: (i, 0)),
        out_shape=jax.ShapeDtypeStruct((n, d_out), jnp.float32),
    )(h, fc1_W.T, fc1_b.reshape(1, -1))

    out = pl.pallas_call(
        _stage3_body,
        grid=(grid,),
        in_specs=[
            pl.BlockSpec((1, d_pool), lambda i: (0, 0)),
            pl.BlockSpec((_BLK, d_out), lambda i: (i, 0)),
            pl.BlockSpec((_NCORES, _BLK, d_pool), lambda i: (0, i, 0)),
            pl.BlockSpec((d_pool, d_out), lambda i: (0, 0)),
            pl.BlockSpec((1, d_out), lambda i: (0, 0)),
        ],
        out_specs=pl.BlockSpec((_BLK, d_out), lambda i: (i, 0)),
        out_shape=jax.ShapeDtypeStruct((n, d_out), jnp.float32),
    )(imu_row, y1, agg3, fc2_W.T, fc2_b.reshape(1, -1))

    return out
